# baseline (speedup 1.0000x reference)
import jax
import jax.numpy as jnp
from jax.experimental import pallas as pl

B, U, C, DK, DV = 16, 64, 1024, 64, 64
RATE = 0.01
UB = 8


def _body(w_ref, kn_ref, vn_ref, r_ref, mk_ref, mv_ref, mr_ref,
          ok_ref, ov_ref, or_ref):
    fresh = w_ref[...] * RATE
    retain = 1.0 - jnp.sum(fresh, axis=0)
    kv = jnp.concatenate([kn_ref[...], vn_ref[...]], axis=2)
    for i in range(UB):
        acc = jax.lax.dot_general(
            fresh[:, i, :], kv[:, i, :],
            dimension_numbers=(((0,), (0,)), ((), ())),
            preferred_element_type=jnp.float32)
        ok_ref[i] = mk_ref[i] * retain[i, :, None] + acc[:, :DK]
        ov_ref[i] = mv_ref[i] * retain[i, :, None] + acc[:, DK:]
    rw = jnp.sum(fresh * r_ref[:].reshape(B, 1, 1), axis=0)
    or_ref[...] = mr_ref[...] * retain + rw


def kernel(weights, key_new, value_new, reward, mem_keys, mem_values, mem_rewards):
    reward2d = reward.reshape(B, 1)
    out_k, out_v, out_r = pl.pallas_call(
        _body,
        grid=(U // UB,),
        in_specs=[
            pl.BlockSpec((B, UB, C), lambda u: (0, u, 0)),
            pl.BlockSpec((B, UB, DK), lambda u: (0, u, 0)),
            pl.BlockSpec((B, UB, DV), lambda u: (0, u, 0)),
            pl.BlockSpec((B, 1), lambda u: (0, 0)),
            pl.BlockSpec((UB, C, DK), lambda u: (u, 0, 0)),
            pl.BlockSpec((UB, C, DV), lambda u: (u, 0, 0)),
            pl.BlockSpec((UB, C), lambda u: (u, 0)),
        ],
        out_specs=[
            pl.BlockSpec((UB, C, DK), lambda u: (u, 0, 0)),
            pl.BlockSpec((UB, C, DV), lambda u: (u, 0, 0)),
            pl.BlockSpec((UB, C), lambda u: (u, 0)),
        ],
        out_shape=[
            jax.ShapeDtypeStruct((U, C, DK), jnp.float32),
            jax.ShapeDtypeStruct((U, C, DV), jnp.float32),
            jax.ShapeDtypeStruct((U, C), jnp.float32),
        ],
    )(weights, key_new, value_new, reward2d, mem_keys, mem_values, mem_rewards)
    return out_k, out_v, out_r
